# baseline (speedup 1.0000x reference)
import functools

import jax
import jax.numpy as jnp
from jax import lax
from jax.experimental import pallas as pl
from jax.experimental.pallas import tpu as pltpu
from jax.experimental.pallas import tpu_sc as plsc

N = 10000
N_PAD = 10112
E_RAW = 160000
E_ALL = E_RAW + N
NC, NS, L = 2, 16, 16
NW = NC * NS
W_E = 5328
E_PAD = NW * W_E
C = 592
CHUNKS = W_E // C
ROWS_PT = N_PAD // NS
F = 64


def _mesh():
    return plsc.VectorSubcoreMesh(
        core_axis_name="c", subcore_axis_name="s", num_cores=NC,
        num_subcores=NS)


def _leaky(v):
    return jnp.where(v >= 0.0, v, 0.2 * v)


def _sc_pass1(H):
    TBL = H * N + L

    @functools.partial(
        pl.kernel,
        out_type=jax.ShapeDtypeStruct((NC, N_PAD, H), jnp.float32),
        mesh=_mesh(),
        compiler_params=pltpu.CompilerParams(needs_layout_passes=False, use_tc_tiling_on_sc=False),
        scratch_types=[
            pltpu.VMEM((TBL,), jnp.float32),
            pltpu.VMEM((TBL,), jnp.float32),
            pltpu.VMEM((C,), jnp.int32),
            pltpu.VMEM((C,), jnp.int32),
            pltpu.VMEM((C, H), jnp.float32),
            pltpu.VMEM_SHARED((N_PAD, H), jnp.float32),
        ],
    )
    def k(src_h, dst_h, als_h, ald_h, zeros_h, out_h,
          als_t, ald_t, src_v, dst_v, p_v, acc):
        cid = lax.axis_index("c")
        sid = lax.axis_index("s")
        wid = sid * NC + cid
        pltpu.sync_copy(zeros_h.at[pl.ds(sid * ROWS_PT, ROWS_PT)],
                        acc.at[pl.ds(sid * ROWS_PT, ROWS_PT)])
        pltpu.sync_copy(als_h, als_t.at[pl.ds(0, H * N)])
        pltpu.sync_copy(ald_h, ald_t.at[pl.ds(0, H * N)])
        als_t[pl.ds(H * N, L)] = jnp.zeros((L,), jnp.float32)
        ald_t[pl.ds(H * N, L)] = jnp.zeros((L,), jnp.float32)
        plsc.subcore_barrier()
        base_e = wid * W_E

        def chunk_body(ci, _):
            pltpu.sync_copy(src_h.at[pl.ds(base_e + ci * C, C)], src_v)
            pltpu.sync_copy(dst_h.at[pl.ds(base_e + ci * C, C)], dst_v)

            def vec_body(i, _):
                s16 = src_v[pl.ds(i * L, L)]
                d16 = dst_v[pl.ds(i * L, L)]
                rows = lax.iota(jnp.int32, L) + i * L
                for h in range(H):
                    a1 = plsc.load_gather(als_t, [s16 + h * N])
                    a2 = plsc.load_gather(ald_t, [d16 + h * N])
                    p = jnp.exp(_leaky(a1 + a2))
                    plsc.store_scatter(
                        p_v, [rows, jnp.full((L,), h, jnp.int32)], p)
                return 0

            lax.fori_loop(0, C // L, vec_body, 0)
            pltpu.sync_copy(p_v, acc.at[dst_v], add=True)
            return 0

        lax.fori_loop(0, CHUNKS, chunk_body, 0)
        plsc.subcore_barrier()
        pltpu.sync_copy(acc.at[pl.ds(sid * ROWS_PT, ROWS_PT)],
                        out_h.at[cid, pl.ds(sid * ROWS_PT, ROWS_PT)])

    return k


C2 = 144
CHUNKS2 = W_E // C2


def _sc_pass2(H):
    TBL = N + L
    RTBL = N_PAD

    @functools.partial(
        pl.kernel,
        out_type=jax.ShapeDtypeStruct((NC, N_PAD, F), jnp.float32),
        mesh=_mesh(),
        compiler_params=pltpu.CompilerParams(needs_layout_passes=False, use_tc_tiling_on_sc=False),
        scratch_types=[
            pltpu.VMEM((TBL,), jnp.float32),
            pltpu.VMEM((TBL,), jnp.float32),
            pltpu.VMEM((RTBL,), jnp.float32),
            pltpu.VMEM((C2,), jnp.int32),
            pltpu.VMEM((C2,), jnp.int32),
            pltpu.VMEM((2, C2), jnp.int32),
            pltpu.VMEM((2, C2), jnp.int32),
            pltpu.VMEM((2, C2), jnp.float32),
            pltpu.VMEM((2, C2, F), jnp.float32),
            pltpu.SemaphoreType.DMA,
            pltpu.SemaphoreType.DMA,
            pltpu.VMEM_SHARED((N_PAD, F), jnp.float32),
        ],
    )
    def k(src_h, dst_h, als_h, ald_h, r_h, xp_h, zeros_h, out_h,
          als_t, ald_t, r_t, src_v, dst_v, sd_v, gi_v, w_v, rows_v,
          sem0, sem1, acc):
        cid = lax.axis_index("c")
        sid = lax.axis_index("s")
        wid = sid * NC + cid
        pltpu.sync_copy(zeros_h.at[pl.ds(sid * ROWS_PT, ROWS_PT)],
                        acc.at[pl.ds(sid * ROWS_PT, ROWS_PT)])
        plsc.subcore_barrier()
        base_e = wid * W_E
        inv = 1.0 / H

        def prep(h, t, b):
            pltpu.sync_copy(src_h.at[pl.ds(base_e + t * C2, C2)], src_v)
            pltpu.sync_copy(dst_h.at[pl.ds(base_e + t * C2, C2)], dst_v)

            def vec_body(i, _):
                s16 = src_v[pl.ds(i * L, L)]
                d16 = dst_v[pl.ds(i * L, L)]
                a1 = plsc.load_gather(als_t, [s16])
                a2 = plsc.load_gather(ald_t, [d16])
                r16 = plsc.load_gather(r_t, [d16])
                p = jnp.exp(_leaky(a1 + a2))
                w_v[b, pl.ds(i * L, L)] = p * r16 * inv
                gi_v[b, pl.ds(i * L, L)] = s16 + h * N
                sd_v[b, pl.ds(i * L, L)] = d16
                return 0

            lax.fori_loop(0, C2 // L, vec_body, 0)

        def kick(b, sem):
            pltpu.async_copy(xp_h.at[gi_v.at[b]], rows_v.at[b], sem)

        def drain(b, sem):
            pltpu.make_async_copy(xp_h.at[gi_v.at[b]], rows_v.at[b],
                                  sem).wait()

        def scale_scatter(b):
            def scale_body(i, _):
                wg = w_v[b, pl.ds(i * L, L)]
                for k in range(L):
                    wb = jnp.full((L,), wg[k], jnp.float32)
                    row = i * L + k
                    for j in range(F // L):
                        rows_v[b, row, pl.ds(j * L, L)] = (
                            rows_v[b, row, pl.ds(j * L, L)] * wb)
                return 0

            lax.fori_loop(0, C2 // L, scale_body, 0)
            pltpu.sync_copy(rows_v.at[b], acc.at[sd_v.at[b]], add=True)

        for h in range(H):
            pltpu.sync_copy(r_h.at[pl.ds(h * N_PAD, N_PAD)], r_t)
            pltpu.sync_copy(als_h.at[pl.ds(h * N, N)], als_t.at[pl.ds(0, N)])
            pltpu.sync_copy(ald_h.at[pl.ds(h * N, N)], ald_t.at[pl.ds(0, N)])
            als_t[pl.ds(N, L)] = jnp.zeros((L,), jnp.float32)
            ald_t[pl.ds(N, L)] = jnp.zeros((L,), jnp.float32)

            prep(h, 0, 0)
            kick(0, sem0)

            def chunk_body(t, _):
                prep(h, 2 * t + 1, 1)
                kick(1, sem1)
                drain(0, sem0)
                scale_scatter(0)
                prep(h, 2 * t + 2, 0)
                kick(0, sem0)
                drain(1, sem1)
                scale_scatter(1)
                return 0

            lax.fori_loop(0, CHUNKS2 // 2, chunk_body, 0)
            drain(0, sem0)
            scale_scatter(0)

        plsc.subcore_barrier()
        pltpu.sync_copy(acc.at[pl.ds(sid * ROWS_PT, ROWS_PT)],
                        out_h.at[cid, pl.ds(sid * ROWS_PT, ROWS_PT)])

    return k


_BLK = 512
_GRID_N = (N + _BLK - 1) // _BLK


def _tc_prep(H, Din, with_parts):
    def body(*refs):
        if with_parts:
            parts, b, w_r, asr_r, adr_r, xp_r, als_r, ald_r = refs
            xb = jnp.maximum(parts[0] + parts[1] + b[...], 0.0)
        else:
            x_r, w_r, asr_r, adr_r, xp_r, als_r, ald_r = refs
            xb = x_r[...]
        xp = jnp.dot(xb, w_r[0], preferred_element_type=jnp.float32)
        xp_r[...] = xp[None]
        als_r[...] = jnp.sum(xp * asr_r[0], axis=1)[None, None]
        ald_r[...] = jnp.sum(xp * adr_r[0], axis=1)[None, None]

    in_specs = []
    if with_parts:
        in_specs.append(pl.BlockSpec((2, _BLK, F), lambda i, h: (0, i, 0)))
        in_specs.append(pl.BlockSpec((1, F), lambda i, h: (0, 0)))
    else:
        in_specs.append(pl.BlockSpec((_BLK, Din), lambda i, h: (i, 0)))
    in_specs += [
        pl.BlockSpec((1, Din, F), lambda i, h: (h, 0, 0)),
        pl.BlockSpec((1, 1, F), lambda i, h: (h, 0, 0)),
        pl.BlockSpec((1, 1, F), lambda i, h: (h, 0, 0)),
    ]
    return pl.pallas_call(
        body,
        grid=(_GRID_N, H),
        in_specs=in_specs,
        out_specs=[
            pl.BlockSpec((1, _BLK, F), lambda i, h: (h, i, 0)),
            pl.BlockSpec((1, 1, _BLK), lambda i, h: (h, 0, i)),
            pl.BlockSpec((1, 1, _BLK), lambda i, h: (h, 0, i)),
        ],
        out_shape=[
            jax.ShapeDtypeStruct((H, N, F), jnp.float32),
            jax.ShapeDtypeStruct((H, 1, N), jnp.float32),
            jax.ShapeDtypeStruct((H, 1, N), jnp.float32),
        ],
    )


def _tc_mid(H):
    grid = (N_PAD + _BLK - 1) // _BLK

    def body(s_r, r_r):
        r_r[...] = jnp.transpose(1.0 / (s_r[0] + s_r[1] + 1e-16))

    return pl.pallas_call(
        body,
        grid=(grid,),
        in_specs=[pl.BlockSpec((2, _BLK, H), lambda i: (0, i, 0))],
        out_specs=pl.BlockSpec((H, _BLK), lambda i: (0, i)),
        out_shape=jax.ShapeDtypeStruct((H, N_PAD), jnp.float32),
    )


def _tc_final():
    def body(parts, b, o_r):
        i = pl.program_id(0)

        @pl.when(i == 0)
        def _():
            o_r[...] = b[...]

        rows = jax.lax.broadcasted_iota(jnp.int32, (_BLK, F), 0) + i * _BLK
        v = jnp.where(rows < N, parts[0] + parts[1], 0.0)
        o_r[...] += jnp.sum(v, axis=0)[None] * (1.0 / N)

    return pl.pallas_call(
        body,
        grid=(_GRID_N,),
        in_specs=[pl.BlockSpec((2, _BLK, F), lambda i: (0, i, 0)),
                  pl.BlockSpec((1, F), lambda i: (0, 0))],
        out_specs=pl.BlockSpec((1, F), lambda i: (0, 0)),
        out_shape=jax.ShapeDtypeStruct((1, F), jnp.float32),
    )


def kernel(x, edge_index, W1, a1_src, a1_dst, b1, W2, a2_src, a2_dst, b2,
           W3, a3_src, a3_dst, b3):
    loops = jnp.arange(N, dtype=edge_index.dtype)
    pad = E_PAD - E_ALL
    src = jnp.concatenate([edge_index[0], loops,
                           jnp.zeros((pad,), edge_index.dtype)])
    dst = jnp.concatenate([edge_index[1], loops,
                           jnp.full((pad,), N, edge_index.dtype)])
    z1 = jnp.zeros((N_PAD, 4), jnp.float32)
    z1b = jnp.zeros((N_PAD, 1), jnp.float32)
    z2 = jnp.zeros((N_PAD, F), jnp.float32)

    def layer(h_in_parts, b_prev, W, a_s, a_d, H, Din, first):
        Wh = W.reshape(Din, H, F).transpose(1, 0, 2)
        a_s = a_s.reshape(H, 1, F)
        a_d = a_d.reshape(H, 1, F)
        if first:
            xp, als, ald = _tc_prep(H, Din, False)(h_in_parts, Wh, a_s, a_d)
        else:
            xp, als, ald = _tc_prep(H, Din, True)(
                h_in_parts, b_prev.reshape(1, F), Wh, a_s, a_d)
        als_f = als.reshape(H * N)
        ald_f = ald.reshape(H * N)
        s_parts = _sc_pass1(H)(src, dst, als_f, ald_f, z1 if H == 4 else z1b)
        r = _tc_mid(H)(s_parts)
        parts = _sc_pass2(H)(src, dst, als_f, ald_f, r.reshape(N_PAD * H),
                             xp.reshape(H * N, F), z2)
        return parts

    p1 = layer(x, None, W1, a1_src, a1_dst, 4, 20, True)
    p2 = layer(p1, b1, W2, a2_src, a2_dst, 4, F, False)
    p3 = layer(p2, b2, W3, a3_src, a3_dst, 1, F, False)
    return _tc_final()(p3, b3.reshape(1, F))


# speedup vs baseline: 1.1152x; 1.1152x over previous
import functools

import jax
import jax.numpy as jnp
from jax import lax
from jax.experimental import pallas as pl
from jax.experimental.pallas import tpu as pltpu
from jax.experimental.pallas import tpu_sc as plsc

N = 10000
N_PAD = 10112
E_RAW = 160000
E_ALL = E_RAW + N
NC, NS, L = 2, 16, 16
NW = NC * NS
W_E = 5328
E_PAD = NW * W_E
C = 592
CHUNKS = W_E // C
C2 = 144
CHUNKS2 = W_E // C2
ROWS_PT = N_PAD // NS
F = 64


def _mesh():
    return plsc.VectorSubcoreMesh(
        core_axis_name="c", subcore_axis_name="s", num_cores=NC,
        num_subcores=NS)


_SC_PARAMS = pltpu.CompilerParams(
    needs_layout_passes=False, use_tc_tiling_on_sc=False)


def _leaky(v):
    return jnp.where(v >= 0.0, v, 0.2 * v)


def _sc_pass1(H):
    TBL = N_PAD * H

    @functools.partial(
        pl.kernel,
        out_type=(jax.ShapeDtypeStruct((NC, N_PAD, H), jnp.float32),
                  jax.ShapeDtypeStruct((E_PAD, H), jnp.float32)),
        mesh=_mesh(),
        compiler_params=_SC_PARAMS,
        scratch_types=[
            pltpu.VMEM((TBL,), jnp.float32),
            pltpu.VMEM((TBL,), jnp.float32),
            pltpu.VMEM((C,), jnp.int32),
            pltpu.VMEM((C,), jnp.int32),
            pltpu.VMEM((C, H), jnp.float32),
            pltpu.VMEM_SHARED((N_PAD, H), jnp.float32),
        ],
    )
    def k(src_h, dst_h, als_h, ald_h, zeros_h, out_h, p_out,
          als_t, ald_t, src_v, dst_v, p_v, acc):
        cid = lax.axis_index("c")
        sid = lax.axis_index("s")
        wid = sid * NC + cid
        pltpu.sync_copy(zeros_h.at[pl.ds(sid * ROWS_PT, ROWS_PT)],
                        acc.at[pl.ds(sid * ROWS_PT, ROWS_PT)])
        pltpu.sync_copy(als_h, als_t)
        pltpu.sync_copy(ald_h, ald_t)
        plsc.subcore_barrier()
        base_e = wid * W_E

        def chunk_body(ci, _):
            pltpu.sync_copy(src_h.at[pl.ds(base_e + ci * C, C)], src_v)
            pltpu.sync_copy(dst_h.at[pl.ds(base_e + ci * C, C)], dst_v)

            def vec_body(i, _):
                s16 = src_v[pl.ds(i * L, L)]
                d16 = dst_v[pl.ds(i * L, L)]
                rows = lax.iota(jnp.int32, L) + i * L
                for h in range(H):
                    a1 = plsc.load_gather(als_t, [s16 * H + h])
                    a2 = plsc.load_gather(ald_t, [d16 * H + h])
                    p = jnp.exp(_leaky(a1 + a2))
                    plsc.store_scatter(
                        p_v, [rows, jnp.full((L,), h, jnp.int32)], p)
                return 0

            lax.fori_loop(0, C // L, vec_body, 0)
            pltpu.sync_copy(p_v, acc.at[dst_v], add=True)
            pltpu.sync_copy(p_v, p_out.at[pl.ds(base_e + ci * C, C)])
            return 0

        lax.fori_loop(0, CHUNKS, chunk_body, 0)
        plsc.subcore_barrier()
        pltpu.sync_copy(acc.at[pl.ds(sid * ROWS_PT, ROWS_PT)],
                        out_h.at[cid, pl.ds(sid * ROWS_PT, ROWS_PT)])

    return k


def _sc_pass2(H):
    RTBL = N_PAD * H
    HF = H * F

    @functools.partial(
        pl.kernel,
        out_type=jax.ShapeDtypeStruct((NC, N_PAD, F), jnp.float32),
        mesh=_mesh(),
        compiler_params=_SC_PARAMS,
        scratch_types=[
            pltpu.VMEM((RTBL,), jnp.float32),
            pltpu.VMEM((C2,), jnp.int32),
            pltpu.VMEM((C2,), jnp.int32),
            pltpu.VMEM((C2 * H,), jnp.float32),
            pltpu.VMEM((C2 * H,), jnp.float32),
            pltpu.VMEM((C2, HF), jnp.float32),
            pltpu.VMEM((C2, F), jnp.float32),
            pltpu.VMEM_SHARED((N_PAD, F), jnp.float32),
            pltpu.SemaphoreType.DMA,
        ],
    )
    def k(src_h, dst_h, p_h, r_h, xp_h, zeros_h, out_h,
          r_t, src_v, dst_v, p_v, w_v, rows_v, comb_v, acc, sem):
        cid = lax.axis_index("c")
        sid = lax.axis_index("s")
        wid = sid * NC + cid
        pltpu.sync_copy(zeros_h.at[pl.ds(sid * ROWS_PT, ROWS_PT)],
                        acc.at[pl.ds(sid * ROWS_PT, ROWS_PT)])
        pltpu.sync_copy(r_h, r_t)
        plsc.subcore_barrier()
        base_e = wid * W_E
        inv = 1.0 / H

        def chunk_body(ci, _):
            pltpu.sync_copy(src_h.at[pl.ds(base_e + ci * C2, C2)], src_v)
            pltpu.sync_copy(dst_h.at[pl.ds(base_e + ci * C2, C2)], dst_v)
            pltpu.sync_copy(p_h.at[pl.ds((base_e + ci * C2) * H, C2 * H)],
                            p_v)
            pltpu.async_copy(xp_h.at[src_v], rows_v, sem)

            def w_body(i, _):
                pos = lax.iota(jnp.int32, L) + i * L
                e16 = lax.shift_right_logical(pos, _HSHIFT)
                h16 = jnp.bitwise_and(pos, H - 1)
                d16 = plsc.load_gather(dst_v, [e16])
                r16 = plsc.load_gather(r_t, [d16 * H + h16])
                p16 = p_v[pl.ds(i * L, L)]
                w_v[pl.ds(i * L, L)] = p16 * r16 * inv
                return 0

            _HSHIFT = {1: 0, 2: 1, 4: 2}[H]
            lax.fori_loop(0, C2 * H // L, w_body, 0)
            pltpu.make_async_copy(xp_h.at[src_v], rows_v, sem).wait()

            EPG = L // H

            def comb_body(i, _):
                wg = w_v[pl.ds(i * L, L)]
                for j in range(EPG):
                    e = i * EPG + j
                    for seg in range(F // L):
                        v = jnp.zeros((L,), jnp.float32)
                        for h in range(H):
                            wb = jnp.full((L,), wg[j * H + h], jnp.float32)
                            v = v + wb * rows_v[e, pl.ds(h * F + seg * L, L)]
                        comb_v[e, pl.ds(seg * L, L)] = v
                return 0

            lax.fori_loop(0, C2 * H // L, comb_body, 0)
            pltpu.sync_copy(comb_v, acc.at[dst_v], add=True)
            return 0

        lax.fori_loop(0, CHUNKS2, chunk_body, 0)
        plsc.subcore_barrier()
        pltpu.sync_copy(acc.at[pl.ds(sid * ROWS_PT, ROWS_PT)],
                        out_h.at[cid, pl.ds(sid * ROWS_PT, ROWS_PT)])

    return k


_BLK = 512
_GRID_N = (N + _BLK - 1) // _BLK


def _tc_prep(H, Din, with_parts):
    HF = H * F

    def body(*refs):
        if with_parts:
            parts, b, w_r, asr_r, adr_r, xp_r, als_r, ald_r = refs
            xb = jnp.maximum(parts[0] + parts[1] + b[...], 0.0)
        else:
            x_r, w_r, asr_r, adr_r, xp_r, als_r, ald_r = refs
            xb = x_r[...]
        xp = jnp.dot(xb, w_r[...], preferred_element_type=jnp.float32)
        xp_r[...] = xp
        als_r[...] = jnp.dot(xp, asr_r[...],
                             preferred_element_type=jnp.float32)
        ald_r[...] = jnp.dot(xp, adr_r[...],
                             preferred_element_type=jnp.float32)

    in_specs = []
    if with_parts:
        in_specs.append(pl.BlockSpec((2, _BLK, F), lambda i: (0, i, 0)))
        in_specs.append(pl.BlockSpec((1, F), lambda i: (0, 0)))
    else:
        in_specs.append(pl.BlockSpec((_BLK, Din), lambda i: (i, 0)))
    in_specs += [
        pl.BlockSpec((Din, HF), lambda i: (0, 0)),
        pl.BlockSpec((HF, H), lambda i: (0, 0)),
        pl.BlockSpec((HF, H), lambda i: (0, 0)),
    ]
    return pl.pallas_call(
        body,
        grid=(_GRID_N,),
        in_specs=in_specs,
        out_specs=[
            pl.BlockSpec((_BLK, HF), lambda i: (i, 0)),
            pl.BlockSpec((_BLK, H), lambda i: (i, 0)),
            pl.BlockSpec((_BLK, H), lambda i: (i, 0)),
        ],
        out_shape=[
            jax.ShapeDtypeStruct((N, HF), jnp.float32),
            jax.ShapeDtypeStruct((N_PAD, H), jnp.float32),
            jax.ShapeDtypeStruct((N_PAD, H), jnp.float32),
        ],
    )


def _tc_mid(H):
    grid = (N_PAD + _BLK - 1) // _BLK

    def body(s_r, r_r):
        r_r[...] = 1.0 / (s_r[0] + s_r[1] + 1e-16)

    return pl.pallas_call(
        body,
        grid=(grid,),
        in_specs=[pl.BlockSpec((2, _BLK, H), lambda i: (0, i, 0))],
        out_specs=pl.BlockSpec((_BLK, H), lambda i: (i, 0)),
        out_shape=jax.ShapeDtypeStruct((N_PAD, H), jnp.float32),
    )


def _tc_final():
    def body(parts, b, o_r):
        i = pl.program_id(0)

        @pl.when(i == 0)
        def _():
            o_r[...] = b[...]

        rows = jax.lax.broadcasted_iota(jnp.int32, (_BLK, F), 0) + i * _BLK
        v = jnp.where(rows < N, parts[0] + parts[1], 0.0)
        o_r[...] += jnp.sum(v, axis=0)[None] * (1.0 / N)

    return pl.pallas_call(
        body,
        grid=(_GRID_N,),
        in_specs=[pl.BlockSpec((2, _BLK, F), lambda i: (0, i, 0)),
                  pl.BlockSpec((1, F), lambda i: (0, 0))],
        out_specs=pl.BlockSpec((1, F), lambda i: (0, 0)),
        out_shape=jax.ShapeDtypeStruct((1, F), jnp.float32),
    )


def kernel(x, edge_index, W1, a1_src, a1_dst, b1, W2, a2_src, a2_dst, b2,
           W3, a3_src, a3_dst, b3):
    loops = jnp.arange(N, dtype=edge_index.dtype)
    pad = E_PAD - E_ALL
    src = jnp.concatenate([edge_index[0], loops,
                           jnp.zeros((pad,), edge_index.dtype)])
    dst = jnp.concatenate([edge_index[1], loops,
                           jnp.full((pad,), N, edge_index.dtype)])
    z1 = jnp.zeros((N_PAD, 4), jnp.float32)
    z1b = jnp.zeros((N_PAD, 1), jnp.float32)
    z2 = jnp.zeros((N_PAD, F), jnp.float32)

    def layer(h_in_parts, b_prev, W, a_s, a_d, H, Din, first):
        eye = jnp.eye(H, dtype=jnp.float32)
        a_s = (eye[:, None, :] * a_s.reshape(H, F)[:, :, None]
               ).reshape(H * F, H)
        a_d = (eye[:, None, :] * a_d.reshape(H, F)[:, :, None]
               ).reshape(H * F, H)
        if first:
            xp, als, ald = _tc_prep(H, Din, False)(h_in_parts, W, a_s, a_d)
        else:
            xp, als, ald = _tc_prep(H, Din, True)(
                h_in_parts, b_prev.reshape(1, F), W, a_s, a_d)
        als_f = als.reshape(N_PAD * H)
        ald_f = ald.reshape(N_PAD * H)
        zp = z1 if H == 4 else z1b
        s_parts, p_e = _sc_pass1(H)(src, dst, als_f, ald_f, zp)
        r = _tc_mid(H)(s_parts)
        parts = _sc_pass2(H)(src, dst, p_e.reshape(E_PAD * H),
                             r.reshape(N_PAD * H), xp, z2)
        return parts

    p1 = layer(x, None, W1, a1_src, a1_dst, 4, 20, True)
    p2 = layer(p1, b1, W2, a2_src, a2_dst, 4, F, False)
    p3 = layer(p2, b2, W3, a3_src, a3_dst, 1, F, False)
    return _tc_final()(p3, b3.reshape(1, F))


# speedup vs baseline: 1.2152x; 1.0897x over previous
import functools

import jax
import jax.numpy as jnp
from jax import lax
from jax.experimental import pallas as pl
from jax.experimental.pallas import tpu as pltpu
from jax.experimental.pallas import tpu_sc as plsc

N = 10000
N_PAD = 10112
E_RAW = 160000
E_ALL = E_RAW + N
NC, NS, L = 2, 16, 16
NW = NC * NS
W_E = 5328
E_PAD = NW * W_E
C = 592
CHUNKS = W_E // C
C2 = 144
CHUNKS2 = W_E // C2
ROWS_PT = N_PAD // NS
F = 64


def _mesh():
    return plsc.VectorSubcoreMesh(
        core_axis_name="c", subcore_axis_name="s", num_cores=NC,
        num_subcores=NS)


_SC_PARAMS = pltpu.CompilerParams(
    needs_layout_passes=False, use_tc_tiling_on_sc=False)


def _leaky(v):
    return jnp.where(v >= 0.0, v, 0.2 * v)


def _sc_pass1(H):
    TBL = N_PAD * H

    @functools.partial(
        pl.kernel,
        out_type=(jax.ShapeDtypeStruct((NC, N_PAD, H), jnp.float32),
                  jax.ShapeDtypeStruct((E_PAD, H), jnp.float32)),
        mesh=_mesh(),
        compiler_params=_SC_PARAMS,
        scratch_types=[
            pltpu.VMEM((TBL,), jnp.float32),
            pltpu.VMEM((TBL,), jnp.float32),
            pltpu.VMEM((C,), jnp.int32),
            pltpu.VMEM((C,), jnp.int32),
            pltpu.VMEM((C, H), jnp.float32),
            pltpu.VMEM_SHARED((N_PAD, H), jnp.float32),
        ],
    )
    def k(src_h, dst_h, als_h, ald_h, zeros_h, out_h, p_out,
          als_t, ald_t, src_v, dst_v, p_v, acc):
        cid = lax.axis_index("c")
        sid = lax.axis_index("s")
        wid = sid * NC + cid
        pltpu.sync_copy(zeros_h.at[pl.ds(sid * ROWS_PT, ROWS_PT)],
                        acc.at[pl.ds(sid * ROWS_PT, ROWS_PT)])
        pltpu.sync_copy(als_h, als_t)
        pltpu.sync_copy(ald_h, ald_t)
        plsc.subcore_barrier()
        base_e = wid * W_E

        def chunk_body(ci, _):
            pltpu.sync_copy(src_h.at[pl.ds(base_e + ci * C, C)], src_v)
            pltpu.sync_copy(dst_h.at[pl.ds(base_e + ci * C, C)], dst_v)

            def vec_body(i, _):
                s16 = src_v[pl.ds(i * L, L)]
                d16 = dst_v[pl.ds(i * L, L)]
                rows = lax.iota(jnp.int32, L) + i * L
                for h in range(H):
                    a1 = plsc.load_gather(als_t, [s16 * H + h])
                    a2 = plsc.load_gather(ald_t, [d16 * H + h])
                    p = jnp.exp(_leaky(a1 + a2))
                    plsc.store_scatter(
                        p_v, [rows, jnp.full((L,), h, jnp.int32)], p)
                return 0

            lax.fori_loop(0, C // L, vec_body, 0)
            pltpu.sync_copy(p_v, acc.at[dst_v], add=True)
            pltpu.sync_copy(p_v, p_out.at[pl.ds(base_e + ci * C, C)])
            return 0

        lax.fori_loop(0, CHUNKS, chunk_body, 0)
        plsc.subcore_barrier()
        pltpu.sync_copy(acc.at[pl.ds(sid * ROWS_PT, ROWS_PT)],
                        out_h.at[cid, pl.ds(sid * ROWS_PT, ROWS_PT)])

    return k


def _sc_pass2(H):
    RTBL = N_PAD * H
    HF = H * F

    @functools.partial(
        pl.kernel,
        out_type=jax.ShapeDtypeStruct((NC, N_PAD, F), jnp.float32),
        mesh=_mesh(),
        compiler_params=_SC_PARAMS,
        scratch_types=[
            pltpu.VMEM((RTBL,), jnp.float32),
            pltpu.VMEM((2, C2), jnp.int32),
            pltpu.VMEM((2, C2), jnp.int32),
            pltpu.VMEM((2, C2 * H), jnp.float32),
            pltpu.VMEM((C2 * H,), jnp.float32),
            pltpu.VMEM((C2, HF), jnp.float32),
            pltpu.VMEM((C2, F), jnp.float32),
            pltpu.VMEM_SHARED((N_PAD, F), jnp.float32),
            pltpu.SemaphoreType.DMA,
            pltpu.SemaphoreType.DMA,
        ],
    )
    def k(src_h, dst_h, p_h, r_h, xp_h, zeros_h, out_h,
          r_t, src_v, dst_v, p_v, w_v, rows_v, comb_v, acc, sem, esem):
        cid = lax.axis_index("c")
        sid = lax.axis_index("s")
        wid = sid * NC + cid
        pltpu.sync_copy(zeros_h.at[pl.ds(sid * ROWS_PT, ROWS_PT)],
                        acc.at[pl.ds(sid * ROWS_PT, ROWS_PT)])
        pltpu.sync_copy(r_h, r_t)
        plsc.subcore_barrier()
        base_e = wid * W_E
        inv = 1.0 / H

        _HSHIFT = {1: 0, 2: 1, 4: 2}[H]
        EPG = L // H

        def eload(t, b):
            pltpu.async_copy(src_h.at[pl.ds(base_e + t * C2, C2)],
                             src_v.at[b], esem)
            pltpu.async_copy(dst_h.at[pl.ds(base_e + t * C2, C2)],
                             dst_v.at[b], esem)
            pltpu.async_copy(p_h.at[pl.ds((base_e + t * C2) * H, C2 * H)],
                             p_v.at[b], esem)

        def ewait(t, b):
            pltpu.make_async_copy(src_h.at[pl.ds(base_e + t * C2, C2)],
                                  src_v.at[b], esem).wait()
            pltpu.make_async_copy(dst_h.at[pl.ds(base_e + t * C2, C2)],
                                  dst_v.at[b], esem).wait()
            pltpu.make_async_copy(p_h.at[pl.ds((base_e + t * C2) * H,
                                               C2 * H)],
                                  p_v.at[b], esem).wait()

        def work(b):
            def w_body(i, _):
                pos = lax.iota(jnp.int32, L) + i * L
                e16 = lax.shift_right_logical(pos, _HSHIFT)
                h16 = jnp.bitwise_and(pos, H - 1)
                d16 = plsc.load_gather(dst_v.at[b], [e16])
                r16 = plsc.load_gather(r_t, [d16 * H + h16])
                p16 = p_v[b, pl.ds(i * L, L)]
                w_v[pl.ds(i * L, L)] = p16 * r16 * inv
                return 0

            lax.fori_loop(0, C2 * H // L, w_body, 0)
            pltpu.make_async_copy(xp_h.at[src_v.at[b]], rows_v, sem).wait()

            def comb_body(i, _):
                wg = w_v[pl.ds(i * L, L)]
                for j in range(EPG):
                    e = i * EPG + j
                    for seg in range(F // L):
                        v = jnp.zeros((L,), jnp.float32)
                        for h in range(H):
                            wb = jnp.full((L,), wg[j * H + h], jnp.float32)
                            v = v + wb * rows_v[e, pl.ds(h * F + seg * L, L)]
                        comb_v[e, pl.ds(seg * L, L)] = v
                return 0

            lax.fori_loop(0, C2 * H // L, comb_body, 0)
            pltpu.sync_copy(comb_v, acc.at[dst_v.at[b]], add=True)

        eload(0, 0)
        ewait(0, 0)
        pltpu.async_copy(xp_h.at[src_v.at[0]], rows_v, sem)
        eload(1, 1)

        def chunk_body(t, _):
            work(0)
            ewait(2 * t + 1, 1)
            pltpu.async_copy(xp_h.at[src_v.at[1]], rows_v, sem)

            @pl.when(2 * t + 2 < CHUNKS2)
            def _():
                eload(2 * t + 2, 0)

            work(1)

            @pl.when(2 * t + 2 < CHUNKS2)
            def _():
                ewait(2 * t + 2, 0)
                pltpu.async_copy(xp_h.at[src_v.at[0]], rows_v, sem)

            @pl.when(2 * t + 3 < CHUNKS2)
            def _():
                eload(2 * t + 3, 1)

            return 0

        lax.fori_loop(0, CHUNKS2 // 2, chunk_body, 0)
        work(0)
        plsc.subcore_barrier()
        pltpu.sync_copy(acc.at[pl.ds(sid * ROWS_PT, ROWS_PT)],
                        out_h.at[cid, pl.ds(sid * ROWS_PT, ROWS_PT)])

    return k


_BLK = 512
_GRID_N = (N + _BLK - 1) // _BLK


def _tc_prep(H, Din, with_parts):
    HF = H * F

    def body(*refs):
        if with_parts:
            parts, b, w_r, asr_r, adr_r, xp_r, als_r, ald_r = refs
            xb = jnp.maximum(parts[0] + parts[1] + b[...], 0.0)
        else:
            x_r, w_r, asr_r, adr_r, xp_r, als_r, ald_r = refs
            xb = x_r[...]
        xp = jnp.dot(xb, w_r[...], preferred_element_type=jnp.float32)
        xp_r[...] = xp
        als_r[...] = jnp.dot(xp, asr_r[...],
                             preferred_element_type=jnp.float32)
        ald_r[...] = jnp.dot(xp, adr_r[...],
                             preferred_element_type=jnp.float32)

    in_specs = []
    if with_parts:
        in_specs.append(pl.BlockSpec((2, _BLK, F), lambda i: (0, i, 0)))
        in_specs.append(pl.BlockSpec((1, F), lambda i: (0, 0)))
    else:
        in_specs.append(pl.BlockSpec((_BLK, Din), lambda i: (i, 0)))
    in_specs += [
        pl.BlockSpec((Din, HF), lambda i: (0, 0)),
        pl.BlockSpec((HF, H), lambda i: (0, 0)),
        pl.BlockSpec((HF, H), lambda i: (0, 0)),
    ]
    return pl.pallas_call(
        body,
        grid=(_GRID_N,),
        in_specs=in_specs,
        out_specs=[
            pl.BlockSpec((_BLK, HF), lambda i: (i, 0)),
            pl.BlockSpec((_BLK, H), lambda i: (i, 0)),
            pl.BlockSpec((_BLK, H), lambda i: (i, 0)),
        ],
        out_shape=[
            jax.ShapeDtypeStruct((N, HF), jnp.float32),
            jax.ShapeDtypeStruct((N_PAD, H), jnp.float32),
            jax.ShapeDtypeStruct((N_PAD, H), jnp.float32),
        ],
    )


def _tc_mid(H):
    grid = (N_PAD + _BLK - 1) // _BLK

    def body(s_r, r_r):
        r_r[...] = 1.0 / (s_r[0] + s_r[1] + 1e-16)

    return pl.pallas_call(
        body,
        grid=(grid,),
        in_specs=[pl.BlockSpec((2, _BLK, H), lambda i: (0, i, 0))],
        out_specs=pl.BlockSpec((_BLK, H), lambda i: (i, 0)),
        out_shape=jax.ShapeDtypeStruct((N_PAD, H), jnp.float32),
    )


def _tc_final():
    def body(parts, b, o_r):
        i = pl.program_id(0)

        @pl.when(i == 0)
        def _():
            o_r[...] = b[...]

        rows = jax.lax.broadcasted_iota(jnp.int32, (_BLK, F), 0) + i * _BLK
        v = jnp.where(rows < N, parts[0] + parts[1], 0.0)
        o_r[...] += jnp.sum(v, axis=0)[None] * (1.0 / N)

    return pl.pallas_call(
        body,
        grid=(_GRID_N,),
        in_specs=[pl.BlockSpec((2, _BLK, F), lambda i: (0, i, 0)),
                  pl.BlockSpec((1, F), lambda i: (0, 0))],
        out_specs=pl.BlockSpec((1, F), lambda i: (0, 0)),
        out_shape=jax.ShapeDtypeStruct((1, F), jnp.float32),
    )


def kernel(x, edge_index, W1, a1_src, a1_dst, b1, W2, a2_src, a2_dst, b2,
           W3, a3_src, a3_dst, b3):
    loops = jnp.arange(N, dtype=edge_index.dtype)
    pad = E_PAD - E_ALL
    src = jnp.concatenate([edge_index[0], loops,
                           jnp.zeros((pad,), edge_index.dtype)])
    dst = jnp.concatenate([edge_index[1], loops,
                           jnp.full((pad,), N, edge_index.dtype)])
    z1 = jnp.zeros((N_PAD, 4), jnp.float32)
    z1b = jnp.zeros((N_PAD, 1), jnp.float32)
    z2 = jnp.zeros((N_PAD, F), jnp.float32)

    def layer(h_in_parts, b_prev, W, a_s, a_d, H, Din, first):
        eye = jnp.eye(H, dtype=jnp.float32)
        a_s = (eye[:, None, :] * a_s.reshape(H, F)[:, :, None]
               ).reshape(H * F, H)
        a_d = (eye[:, None, :] * a_d.reshape(H, F)[:, :, None]
               ).reshape(H * F, H)
        if first:
            xp, als, ald = _tc_prep(H, Din, False)(h_in_parts, W, a_s, a_d)
        else:
            xp, als, ald = _tc_prep(H, Din, True)(
                h_in_parts, b_prev.reshape(1, F), W, a_s, a_d)
        als_f = als.reshape(N_PAD * H)
        ald_f = ald.reshape(N_PAD * H)
        zp = z1 if H == 4 else z1b
        s_parts, p_e = _sc_pass1(H)(src, dst, als_f, ald_f, zp)
        r = _tc_mid(H)(s_parts)
        parts = _sc_pass2(H)(src, dst, p_e.reshape(E_PAD * H),
                             r.reshape(N_PAD * H), xp, z2)
        return parts

    p1 = layer(x, None, W1, a1_src, a1_dst, 4, 20, True)
    p2 = layer(p1, b1, W2, a2_src, a2_dst, 4, F, False)
    p3 = layer(p2, b2, W3, a3_src, a3_dst, 1, F, False)
    return _tc_final()(p3, b3.reshape(1, F))


# speedup vs baseline: 1.2349x; 1.0162x over previous
import functools

import jax
import jax.numpy as jnp
from jax import lax
from jax.experimental import pallas as pl
from jax.experimental.pallas import tpu as pltpu
from jax.experimental.pallas import tpu_sc as plsc

N = 10000
N_PAD = 10112
E_RAW = 160000
E_ALL = E_RAW + N
NC, NS, L = 2, 16, 16
NW = NC * NS
W_E = 5328
E_PAD = NW * W_E
C = 592
CHUNKS = W_E // C
C2 = 144
CHUNKS2 = W_E // C2
ROWS_PT = N_PAD // NS
F = 64


def _mesh():
    return plsc.VectorSubcoreMesh(
        core_axis_name="c", subcore_axis_name="s", num_cores=NC,
        num_subcores=NS)


_SC_PARAMS = pltpu.CompilerParams(
    needs_layout_passes=False, use_tc_tiling_on_sc=False)


def _leaky(v):
    return jnp.where(v >= 0.0, v, 0.2 * v)


def _sc_pass1(H):
    TBL = N_PAD * H

    @functools.partial(
        pl.kernel,
        out_type=(jax.ShapeDtypeStruct((NC, N_PAD, H), jnp.float32),
                  jax.ShapeDtypeStruct((E_PAD, H), jnp.float32)),
        mesh=_mesh(),
        compiler_params=_SC_PARAMS,
        scratch_types=[
            pltpu.VMEM((TBL,), jnp.float32),
            pltpu.VMEM((TBL,), jnp.float32),
            pltpu.VMEM((2, C), jnp.int32),
            pltpu.VMEM((2, C), jnp.int32),
            pltpu.VMEM((C, H), jnp.float32),
            pltpu.VMEM_SHARED((N_PAD, H), jnp.float32),
            pltpu.SemaphoreType.DMA,
        ],
    )
    def k(src_h, dst_h, als_h, ald_h, zeros_h, out_h, p_out,
          als_t, ald_t, src_v, dst_v, p_v, acc, esem):
        cid = lax.axis_index("c")
        sid = lax.axis_index("s")
        wid = sid * NC + cid
        pltpu.sync_copy(zeros_h.at[pl.ds(sid * ROWS_PT, ROWS_PT)],
                        acc.at[pl.ds(sid * ROWS_PT, ROWS_PT)])
        pltpu.sync_copy(als_h, als_t)
        pltpu.sync_copy(ald_h, ald_t)
        plsc.subcore_barrier()
        base_e = wid * W_E

        def eload(t, b):
            pltpu.async_copy(src_h.at[pl.ds(base_e + t * C, C)],
                             src_v.at[b], esem)
            pltpu.async_copy(dst_h.at[pl.ds(base_e + t * C, C)],
                             dst_v.at[b], esem)

        def ewait(t, b):
            pltpu.make_async_copy(src_h.at[pl.ds(base_e + t * C, C)],
                                  src_v.at[b], esem).wait()
            pltpu.make_async_copy(dst_h.at[pl.ds(base_e + t * C, C)],
                                  dst_v.at[b], esem).wait()

        def work(t, b):
            def vec_body(i, _):
                s16 = src_v[b, pl.ds(i * L, L)]
                d16 = dst_v[b, pl.ds(i * L, L)]
                rows = lax.iota(jnp.int32, L) + i * L
                for h in range(H):
                    a1 = plsc.load_gather(als_t, [s16 * H + h])
                    a2 = plsc.load_gather(ald_t, [d16 * H + h])
                    p = jnp.exp(_leaky(a1 + a2))
                    plsc.store_scatter(
                        p_v, [rows, jnp.full((L,), h, jnp.int32)], p)
                return 0

            lax.fori_loop(0, C // L, vec_body, 0)
            pltpu.sync_copy(p_v, acc.at[dst_v.at[b]], add=True)
            pltpu.sync_copy(p_v, p_out.at[pl.ds(base_e + t * C, C)])

        eload(0, 0)
        ewait(0, 0)
        eload(1, 1)

        def chunk_body(u, _):
            work(2 * u, 0)
            ewait(2 * u + 1, 1)

            @pl.when(2 * u + 2 < CHUNKS)
            def _():
                eload(2 * u + 2, 0)

            work(2 * u + 1, 1)

            @pl.when(2 * u + 2 < CHUNKS)
            def _():
                ewait(2 * u + 2, 0)

            @pl.when(2 * u + 3 < CHUNKS)
            def _():
                eload(2 * u + 3, 1)

            return 0

        lax.fori_loop(0, CHUNKS // 2, chunk_body, 0)
        work(CHUNKS - 1, 0)
        plsc.subcore_barrier()
        pltpu.sync_copy(acc.at[pl.ds(sid * ROWS_PT, ROWS_PT)],
                        out_h.at[cid, pl.ds(sid * ROWS_PT, ROWS_PT)])

    return k


def _sc_pass2(H):
    RTBL = N_PAD * H
    HF = H * F

    @functools.partial(
        pl.kernel,
        out_type=jax.ShapeDtypeStruct((NC, N_PAD, F), jnp.float32),
        mesh=_mesh(),
        compiler_params=_SC_PARAMS,
        scratch_types=[
            pltpu.VMEM((RTBL,), jnp.float32),
            pltpu.VMEM((2, C2), jnp.int32),
            pltpu.VMEM((2, C2), jnp.int32),
            pltpu.VMEM((2, C2 * H), jnp.float32),
            pltpu.VMEM((C2 * H,), jnp.float32),
            pltpu.VMEM((C2, HF), jnp.float32),
            pltpu.VMEM((C2, F), jnp.float32),
            pltpu.VMEM_SHARED((N_PAD, F), jnp.float32),
            pltpu.SemaphoreType.DMA,
            pltpu.SemaphoreType.DMA,
        ],
    )
    def k(src_h, dst_h, p_h, r_h, xp_h, zeros_h, out_h,
          r_t, src_v, dst_v, p_v, w_v, rows_v, comb_v, acc, sem, esem):
        cid = lax.axis_index("c")
        sid = lax.axis_index("s")
        wid = sid * NC + cid
        pltpu.sync_copy(zeros_h.at[pl.ds(sid * ROWS_PT, ROWS_PT)],
                        acc.at[pl.ds(sid * ROWS_PT, ROWS_PT)])
        pltpu.sync_copy(r_h, r_t)
        plsc.subcore_barrier()
        base_e = wid * W_E
        inv = 1.0 / H

        _HSHIFT = {1: 0, 2: 1, 4: 2}[H]
        EPG = L // H

        def eload(t, b):
            pltpu.async_copy(src_h.at[pl.ds(base_e + t * C2, C2)],
                             src_v.at[b], esem)
            pltpu.async_copy(dst_h.at[pl.ds(base_e + t * C2, C2)],
                             dst_v.at[b], esem)
            pltpu.async_copy(p_h.at[pl.ds((base_e + t * C2) * H, C2 * H)],
                             p_v.at[b], esem)

        def ewait(t, b):
            pltpu.make_async_copy(src_h.at[pl.ds(base_e + t * C2, C2)],
                                  src_v.at[b], esem).wait()
            pltpu.make_async_copy(dst_h.at[pl.ds(base_e + t * C2, C2)],
                                  dst_v.at[b], esem).wait()
            pltpu.make_async_copy(p_h.at[pl.ds((base_e + t * C2) * H,
                                               C2 * H)],
                                  p_v.at[b], esem).wait()

        def work(b):
            def w_body(i, _):
                pos = lax.iota(jnp.int32, L) + i * L
                e16 = lax.shift_right_logical(pos, _HSHIFT)
                h16 = jnp.bitwise_and(pos, H - 1)
                d16 = plsc.load_gather(dst_v.at[b], [e16])
                r16 = plsc.load_gather(r_t, [d16 * H + h16])
                p16 = p_v[b, pl.ds(i * L, L)]
                w_v[pl.ds(i * L, L)] = p16 * r16 * inv
                return 0

            lax.fori_loop(0, C2 * H // L, w_body, 0)
            pltpu.make_async_copy(xp_h.at[src_v.at[b]], rows_v, sem).wait()

            def comb_body(i, _):
                wg = w_v[pl.ds(i * L, L)]
                for j in range(EPG):
                    e = i * EPG + j
                    for seg in range(F // L):
                        v = jnp.zeros((L,), jnp.float32)
                        for h in range(H):
                            wb = jnp.full((L,), wg[j * H + h], jnp.float32)
                            v = v + wb * rows_v[e, pl.ds(h * F + seg * L, L)]
                        comb_v[e, pl.ds(seg * L, L)] = v
                return 0

            lax.fori_loop(0, C2 * H // L, comb_body, 0)
            pltpu.sync_copy(comb_v, acc.at[dst_v.at[b]], add=True)

        eload(0, 0)
        ewait(0, 0)
        pltpu.async_copy(xp_h.at[src_v.at[0]], rows_v, sem)
        eload(1, 1)

        def chunk_body(t, _):
            work(0)
            ewait(2 * t + 1, 1)
            pltpu.async_copy(xp_h.at[src_v.at[1]], rows_v, sem)

            @pl.when(2 * t + 2 < CHUNKS2)
            def _():
                eload(2 * t + 2, 0)

            work(1)

            @pl.when(2 * t + 2 < CHUNKS2)
            def _():
                ewait(2 * t + 2, 0)
                pltpu.async_copy(xp_h.at[src_v.at[0]], rows_v, sem)

            @pl.when(2 * t + 3 < CHUNKS2)
            def _():
                eload(2 * t + 3, 1)

            return 0

        lax.fori_loop(0, CHUNKS2 // 2, chunk_body, 0)
        work(0)
        plsc.subcore_barrier()
        pltpu.sync_copy(acc.at[pl.ds(sid * ROWS_PT, ROWS_PT)],
                        out_h.at[cid, pl.ds(sid * ROWS_PT, ROWS_PT)])

    return k


_BLK = 512
_GRID_N = (N + _BLK - 1) // _BLK


def _tc_prep(H, Din, with_parts):
    HF = H * F

    def body(*refs):
        if with_parts:
            parts, b, w_r, asr_r, adr_r, xp_r, als_r, ald_r = refs
            xb = jnp.maximum(parts[0] + parts[1] + b[...], 0.0)
        else:
            x_r, w_r, asr_r, adr_r, xp_r, als_r, ald_r = refs
            xb = x_r[...]
        xp = jnp.dot(xb, w_r[...], preferred_element_type=jnp.float32)
        xp_r[...] = xp
        als_r[...] = jnp.dot(xp, asr_r[...],
                             preferred_element_type=jnp.float32)
        ald_r[...] = jnp.dot(xp, adr_r[...],
                             preferred_element_type=jnp.float32)

    in_specs = []
    if with_parts:
        in_specs.append(pl.BlockSpec((2, _BLK, F), lambda i: (0, i, 0)))
        in_specs.append(pl.BlockSpec((1, F), lambda i: (0, 0)))
    else:
        in_specs.append(pl.BlockSpec((_BLK, Din), lambda i: (i, 0)))
    in_specs += [
        pl.BlockSpec((Din, HF), lambda i: (0, 0)),
        pl.BlockSpec((HF, H), lambda i: (0, 0)),
        pl.BlockSpec((HF, H), lambda i: (0, 0)),
    ]
    return pl.pallas_call(
        body,
        grid=(_GRID_N,),
        in_specs=in_specs,
        out_specs=[
            pl.BlockSpec((_BLK, HF), lambda i: (i, 0)),
            pl.BlockSpec((_BLK, H), lambda i: (i, 0)),
            pl.BlockSpec((_BLK, H), lambda i: (i, 0)),
        ],
        out_shape=[
            jax.ShapeDtypeStruct((N, HF), jnp.float32),
            jax.ShapeDtypeStruct((N_PAD, H), jnp.float32),
            jax.ShapeDtypeStruct((N_PAD, H), jnp.float32),
        ],
    )


def _tc_mid(H):
    grid = (N_PAD + _BLK - 1) // _BLK

    def body(s_r, r_r):
        r_r[...] = 1.0 / (s_r[0] + s_r[1] + 1e-16)

    return pl.pallas_call(
        body,
        grid=(grid,),
        in_specs=[pl.BlockSpec((2, _BLK, H), lambda i: (0, i, 0))],
        out_specs=pl.BlockSpec((_BLK, H), lambda i: (i, 0)),
        out_shape=jax.ShapeDtypeStruct((N_PAD, H), jnp.float32),
    )


def _tc_final():
    def body(parts, b, o_r):
        i = pl.program_id(0)

        @pl.when(i == 0)
        def _():
            o_r[...] = b[...]

        rows = jax.lax.broadcasted_iota(jnp.int32, (_BLK, F), 0) + i * _BLK
        v = jnp.where(rows < N, parts[0] + parts[1], 0.0)
        o_r[...] += jnp.sum(v, axis=0)[None] * (1.0 / N)

    return pl.pallas_call(
        body,
        grid=(_GRID_N,),
        in_specs=[pl.BlockSpec((2, _BLK, F), lambda i: (0, i, 0)),
                  pl.BlockSpec((1, F), lambda i: (0, 0))],
        out_specs=pl.BlockSpec((1, F), lambda i: (0, 0)),
        out_shape=jax.ShapeDtypeStruct((1, F), jnp.float32),
    )


def kernel(x, edge_index, W1, a1_src, a1_dst, b1, W2, a2_src, a2_dst, b2,
           W3, a3_src, a3_dst, b3):
    loops = jnp.arange(N, dtype=edge_index.dtype)
    pad = E_PAD - E_ALL
    src = jnp.concatenate([edge_index[0], loops,
                           jnp.zeros((pad,), edge_index.dtype)])
    dst = jnp.concatenate([edge_index[1], loops,
                           jnp.full((pad,), N, edge_index.dtype)])
    z1 = jnp.zeros((N_PAD, 4), jnp.float32)
    z1b = jnp.zeros((N_PAD, 1), jnp.float32)
    z2 = jnp.zeros((N_PAD, F), jnp.float32)

    def layer(h_in_parts, b_prev, W, a_s, a_d, H, Din, first):
        eye = jnp.eye(H, dtype=jnp.float32)
        a_s = (eye[:, None, :] * a_s.reshape(H, F)[:, :, None]
               ).reshape(H * F, H)
        a_d = (eye[:, None, :] * a_d.reshape(H, F)[:, :, None]
               ).reshape(H * F, H)
        if first:
            xp, als, ald = _tc_prep(H, Din, False)(h_in_parts, W, a_s, a_d)
        else:
            xp, als, ald = _tc_prep(H, Din, True)(
                h_in_parts, b_prev.reshape(1, F), W, a_s, a_d)
        als_f = als.reshape(N_PAD * H)
        ald_f = ald.reshape(N_PAD * H)
        zp = z1 if H == 4 else z1b
        s_parts, p_e = _sc_pass1(H)(src, dst, als_f, ald_f, zp)
        r = _tc_mid(H)(s_parts)
        parts = _sc_pass2(H)(src, dst, p_e.reshape(E_PAD * H),
                             r.reshape(N_PAD * H), xp, z2)
        return parts

    p1 = layer(x, None, W1, a1_src, a1_dst, 4, 20, True)
    p2 = layer(p1, b1, W2, a2_src, a2_dst, 4, F, False)
    p3 = layer(p2, b2, W3, a3_src, a3_dst, 1, F, False)
    return _tc_final()(p3, b3.reshape(1, F))
